# baseline (speedup 1.0000x reference)
import functools

import jax
import jax.numpy as jnp
from jax import lax
from jax.experimental import pallas as pl
from jax.experimental.pallas import tpu as pltpu
from jax.experimental.pallas import tpu_sc as plsc

N = 10000
E = 320000
D = 128
H = 128
C = 6

FT = 4
CE = 16000
NCH = E // CE
GP = CE // 16
ES = E // 16

_mesh = plsc.VectorSubcoreMesh(core_axis_name="c", subcore_axis_name="s")


@functools.partial(
    pl.kernel,
    mesh=_mesh,
    compiler_params=pltpu.CompilerParams(needs_layout_passes=False, use_tc_tiling_on_sc=False, disable_bounds_checks=True),
    out_type=jax.ShapeDtypeStruct((H, N), jnp.float32),
    scratch_types=(
        [pltpu.VMEM((N,), jnp.int32) for _ in range(2)]
        + [pltpu.VMEM((N,), jnp.float32) for _ in range(FT)]
        + [
            pltpu.VMEM((CE,), jnp.int32),
            pltpu.VMEM((CE,), jnp.int32),
            pltpu.SemaphoreType.DMA,
            pltpu.SemaphoreType.DMA,
            pltpu.SemaphoreType.DMA,
        ]
    ),
)
def _agg_h(hp, pe, out, h0, h1, a0, a1, a2, a3, eb0, eb1, s0, s1, st):
    hb = (h0, h1)
    ac = (a0, a1, a2, a3)
    wid = lax.axis_index("c") * 16 + lax.axis_index("s")
    base = wid * FT

    pltpu.async_copy(pe.at[pl.ds(0, CE)], eb0, s0)
    pltpu.async_copy(pe.at[pl.ds(CE, CE)], eb1, s1)
    c0 = pltpu.async_copy(hp.at[2 * wid], h0, st)
    c1 = pltpu.async_copy(hp.at[2 * wid + 1], h1, st)

    @plsc.parallel_loop(0, N // 16, unroll=8)
    def _zero(j):
        z = jnp.zeros((16,), jnp.float32)
        for f in range(FT):
            ac[f][pl.ds(j * 16, 16)] = z

    c0.wait()
    c1.wait()

    def process(eb):
        @plsc.parallel_loop(0, GP, unroll=8)
        def _gather_scatter(g):
            pe16 = eb[pl.ds(g * 16, 16)]
            src = lax.shift_right_logical(pe16, 16)
            dst = jnp.bitwise_and(pe16, jnp.int32(0xFFFF))
            for p in range(2):
                v = plsc.load_gather(hb[p], [src])
                lo = plsc.bitcast(lax.shift_left(v, 16), jnp.float32)
                hi = plsc.bitcast(
                    jnp.bitwise_and(v, jnp.int32(-65536)), jnp.float32)
                plsc.addupdate_scatter(ac[2 * p], [dst], lo)
                plsc.addupdate_scatter(ac[2 * p + 1], [dst], hi)

    def outer(i, _):
        c = 2 * i
        pltpu.make_async_copy(pe.at[pl.ds(0, CE)], eb0, s0).wait()
        process(eb0)

        @pl.when(c + 2 < NCH)
        def _():
            pltpu.async_copy(pe.at[pl.ds((c + 2) * CE, CE)], eb0, s0)

        pltpu.make_async_copy(pe.at[pl.ds(0, CE)], eb1, s1).wait()
        process(eb1)

        @pl.when(c + 3 < NCH)
        def _():
            pltpu.async_copy(pe.at[pl.ds((c + 3) * CE, CE)], eb1, s1)

        return 0

    lax.fori_loop(0, NCH // 2, outer, 0)
    for f in range(FT):
        pltpu.sync_copy(ac[f], out.at[base + f])


@functools.partial(
    pl.kernel,
    mesh=_mesh,
    compiler_params=pltpu.CompilerParams(needs_layout_passes=False, use_tc_tiling_on_sc=False, disable_bounds_checks=True),
    out_type=jax.ShapeDtypeStruct((32, 3, N), jnp.float32),
    scratch_types=(
        [pltpu.VMEM((N,), jnp.float32) for _ in range(3)]
        + [pltpu.VMEM((N,), jnp.float32) for _ in range(3)]
        + [pltpu.VMEM((ES,), jnp.int32)]
    ),
)
def _agg_out(h3t, pe, out, h0, h1, h2, a0, a1, a2, eb):
    hb = (h0, h1, h2)
    ac = (a0, a1, a2)
    wid = lax.axis_index("c") * 16 + lax.axis_index("s")
    grp = wid // 16
    slot = wid % 16

    pltpu.sync_copy(pe.at[pl.ds(slot * ES, ES)], eb)
    for f in range(3):
        pltpu.sync_copy(h3t.at[grp * 3 + f], hb[f])

    @plsc.parallel_loop(0, N // 16, unroll=8)
    def _zero(j):
        z = jnp.zeros((16,), jnp.float32)
        for f in range(3):
            ac[f][pl.ds(j * 16, 16)] = z

    @plsc.parallel_loop(0, ES // 16, unroll=8)
    def _gather_scatter(g):
        pe16 = eb[pl.ds(g * 16, 16)]
        src = lax.shift_right_logical(pe16, 16)
        dst = jnp.bitwise_and(pe16, jnp.int32(0xFFFF))
        for f in range(3):
            vals = plsc.load_gather(hb[f], [src])
            plsc.addupdate_scatter(ac[f], [dst], vals)

    for f in range(3):
        pltpu.sync_copy(ac[f], out.at[wid, f])


def _pack_pairs(ye, yo):
    ue = lax.bitcast_convert_type(
        ye.astype(jnp.bfloat16), jnp.uint16).astype(jnp.uint32)
    uo = lax.bitcast_convert_type(
        yo.astype(jnp.bfloat16), jnp.uint16).astype(jnp.uint32)
    return lax.bitcast_convert_type(
        jnp.bitwise_or(ue, lax.shift_left(uo, jnp.uint32(16))), jnp.int32)


def _mm1_pack_body(e_ref, x_ref, we_ref, wo_ref, pe_ref, o_ref):
    pe_ref[...] = jnp.bitwise_or(
        lax.shift_left(e_ref[0, :], 16), e_ref[1, :])
    dims = (((0,), (1,)), ((), ()))
    ye = lax.dot_general(we_ref[...], x_ref[...], dims,
                         preferred_element_type=jnp.float32)
    yo = lax.dot_general(wo_ref[...], x_ref[...], dims,
                         preferred_element_type=jnp.float32)
    o_ref[...] = _pack_pairs(ye, yo)


def _mm1_pack(edge_index, x, we, wo):
    return pl.pallas_call(
        _mm1_pack_body,
        out_shape=(jax.ShapeDtypeStruct((E,), jnp.int32),
                   jax.ShapeDtypeStruct((H // 2, N), jnp.int32)),
    )(edge_index, x, we, wo)


def _mm_t_body(h_ref, w_ref, o_ref):
    o_ref[...] = lax.dot_general(
        w_ref[...], h_ref[...], (((0,), (0,)), ((), ())),
        preferred_element_type=jnp.float32)


def _mm_t(ht, w):
    do = w.shape[1]
    return pl.pallas_call(
        _mm_t_body,
        out_shape=jax.ShapeDtypeStruct((do, N), jnp.float32),
    )(ht, w)


def _mm_t_pack_body(h_ref, we_ref, wo_ref, o_ref):
    dims = (((0,), (0,)), ((), ()))
    ye = lax.dot_general(we_ref[...], h_ref[...], dims,
                         preferred_element_type=jnp.float32)
    yo = lax.dot_general(wo_ref[...], h_ref[...], dims,
                         preferred_element_type=jnp.float32)
    o_ref[...] = _pack_pairs(ye, yo)


def _mm_t_pack(ht, we, wo):
    do2 = we.shape[1]
    return pl.pallas_call(
        _mm_t_pack_body,
        out_shape=jax.ShapeDtypeStruct((do2, N), jnp.int32),
    )(ht, we, wo)


def _comb_body(p_ref, o_ref):
    p = p_ref[...]
    s0 = jnp.sum(p[0:16], axis=0)
    s1 = jnp.sum(p[16:32], axis=0)
    s6 = jnp.concatenate([s0, s1], axis=0)
    i6 = (lax.broadcasted_iota(jnp.int32, (C, C), 0)
          == lax.broadcasted_iota(jnp.int32, (C, C), 1)).astype(jnp.float32)
    o_ref[...] = lax.dot_general(
        s6, i6, (((0,), (0,)), ((), ())), preferred_element_type=jnp.float32)


def _combine(parts):
    return pl.pallas_call(
        _comb_body,
        out_shape=jax.ShapeDtypeStruct((N, C), jnp.float32),
    )(parts)


def kernel(x, edge_index, W1, W2, W3):
    pe, h1p = _mm1_pack(edge_index, x, W1[:, 0::2], W1[:, 1::2])
    a1t = _agg_h(h1p, pe)
    h2p = _mm_t_pack(a1t, W2[:, 0::2], W2[:, 1::2])
    a2t = _agg_h(h2p, pe)
    h3t = _mm_t(a2t, W3)
    parts = _agg_out(h3t, pe)
    return _combine(parts)


# speedup vs baseline: 1.0164x; 1.0164x over previous
import functools

import jax
import jax.numpy as jnp
from jax import lax
from jax.experimental import pallas as pl
from jax.experimental.pallas import tpu as pltpu
from jax.experimental.pallas import tpu_sc as plsc

N = 10000
E = 320000
D = 128
H = 128
C = 6

FT = 4
CE = 6400
NCH = E // CE
GP = CE // 16
ES = E // 16

_mesh = plsc.VectorSubcoreMesh(core_axis_name="c", subcore_axis_name="s")


@functools.partial(
    pl.kernel,
    mesh=_mesh,
    compiler_params=pltpu.CompilerParams(needs_layout_passes=False, use_tc_tiling_on_sc=False, disable_bounds_checks=True),
    out_type=jax.ShapeDtypeStruct((H, N), jnp.float32),
    scratch_types=(
        [pltpu.VMEM((N,), jnp.int32) for _ in range(2)]
        + [pltpu.VMEM((N,), jnp.float32) for _ in range(FT)]
        + [
            pltpu.VMEM((CE,), jnp.int32),
            pltpu.VMEM((CE,), jnp.int32),
            pltpu.SemaphoreType.DMA,
            pltpu.SemaphoreType.DMA,
            pltpu.SemaphoreType.DMA,
        ]
    ),
)
def _agg_h(hp, pe, out, h0, h1, a0, a1, a2, a3, eb0, eb1, s0, s1, st):
    hb = (h0, h1)
    ac = (a0, a1, a2, a3)
    wid = lax.axis_index("c") * 16 + lax.axis_index("s")
    base = wid * FT

    pltpu.async_copy(pe.at[pl.ds(0, CE)], eb0, s0)
    pltpu.async_copy(pe.at[pl.ds(CE, CE)], eb1, s1)
    c0 = pltpu.async_copy(hp.at[2 * wid], h0, st)
    c1 = pltpu.async_copy(hp.at[2 * wid + 1], h1, st)

    @plsc.parallel_loop(0, N // 16, unroll=8)
    def _zero(j):
        z = jnp.zeros((16,), jnp.float32)
        for f in range(FT):
            ac[f][pl.ds(j * 16, 16)] = z

    c0.wait()
    c1.wait()

    def process(eb):
        @plsc.parallel_loop(0, GP, unroll=8)
        def _gather_scatter(g):
            pe16 = eb[pl.ds(g * 16, 16)]
            src = lax.shift_right_logical(pe16, 16)
            dst = jnp.bitwise_and(pe16, jnp.int32(0xFFFF))
            for p in range(2):
                v = plsc.load_gather(hb[p], [src])
                lo = plsc.bitcast(lax.shift_left(v, 16), jnp.float32)
                hi = plsc.bitcast(
                    jnp.bitwise_and(v, jnp.int32(-65536)), jnp.float32)
                plsc.addupdate_scatter(ac[2 * p], [dst], lo)
                plsc.addupdate_scatter(ac[2 * p + 1], [dst], hi)

    def outer(i, _):
        c = 2 * i
        pltpu.make_async_copy(pe.at[pl.ds(0, CE)], eb0, s0).wait()
        process(eb0)

        @pl.when(c + 2 < NCH)
        def _():
            pltpu.async_copy(pe.at[pl.ds((c + 2) * CE, CE)], eb0, s0)

        pltpu.make_async_copy(pe.at[pl.ds(0, CE)], eb1, s1).wait()
        process(eb1)

        @pl.when(c + 3 < NCH)
        def _():
            pltpu.async_copy(pe.at[pl.ds((c + 3) * CE, CE)], eb1, s1)

        return 0

    lax.fori_loop(0, NCH // 2, outer, 0)
    wb = [pltpu.async_copy(ac[f], out.at[base + f], st) for f in range(FT)]
    for w in wb:
        w.wait()


@functools.partial(
    pl.kernel,
    mesh=_mesh,
    compiler_params=pltpu.CompilerParams(needs_layout_passes=False, use_tc_tiling_on_sc=False, disable_bounds_checks=True),
    out_type=jax.ShapeDtypeStruct((32, 3, N), jnp.float32),
    scratch_types=(
        [pltpu.VMEM((N,), jnp.float32) for _ in range(3)]
        + [pltpu.VMEM((N,), jnp.float32) for _ in range(3)]
        + [pltpu.VMEM((ES,), jnp.int32), pltpu.SemaphoreType.DMA]
    ),
)
def _agg_out(h3t, pe, out, h0, h1, h2, a0, a1, a2, eb, st):
    hb = (h0, h1, h2)
    ac = (a0, a1, a2)
    wid = lax.axis_index("c") * 16 + lax.axis_index("s")
    grp = wid // 16
    slot = wid % 16

    cs = [pltpu.async_copy(pe.at[pl.ds(slot * ES, ES)], eb, st)]
    cs += [pltpu.async_copy(h3t.at[grp * 3 + f], hb[f], st) for f in range(3)]

    @plsc.parallel_loop(0, N // 16, unroll=8)
    def _zero(j):
        z = jnp.zeros((16,), jnp.float32)
        for f in range(3):
            ac[f][pl.ds(j * 16, 16)] = z

    for c in cs:
        c.wait()

    @plsc.parallel_loop(0, ES // 16, unroll=8)
    def _gather_scatter(g):
        pe16 = eb[pl.ds(g * 16, 16)]
        src = lax.shift_right_logical(pe16, 16)
        dst = jnp.bitwise_and(pe16, jnp.int32(0xFFFF))
        for f in range(3):
            vals = plsc.load_gather(hb[f], [src])
            plsc.addupdate_scatter(ac[f], [dst], vals)

    wb = [pltpu.async_copy(ac[f], out.at[wid, f], st) for f in range(3)]
    for w in wb:
        w.wait()


def _pack_pairs(ye, yo):
    ue = lax.bitcast_convert_type(
        ye.astype(jnp.bfloat16), jnp.uint16).astype(jnp.uint32)
    uo = lax.bitcast_convert_type(
        yo.astype(jnp.bfloat16), jnp.uint16).astype(jnp.uint32)
    return lax.bitcast_convert_type(
        jnp.bitwise_or(ue, lax.shift_left(uo, jnp.uint32(16))), jnp.int32)


def _mm1_pack_body(e_ref, x_ref, we_ref, wo_ref, pe_ref, o_ref):
    pe_ref[...] = jnp.bitwise_or(
        lax.shift_left(e_ref[0, :], 16), e_ref[1, :])
    dims = (((0,), (1,)), ((), ()))
    ye = lax.dot_general(we_ref[...], x_ref[...], dims,
                         preferred_element_type=jnp.float32)
    yo = lax.dot_general(wo_ref[...], x_ref[...], dims,
                         preferred_element_type=jnp.float32)
    o_ref[...] = _pack_pairs(ye, yo)


def _mm1_pack(edge_index, x, we, wo):
    return pl.pallas_call(
        _mm1_pack_body,
        out_shape=(jax.ShapeDtypeStruct((E,), jnp.int32),
                   jax.ShapeDtypeStruct((H // 2, N), jnp.int32)),
    )(edge_index, x, we, wo)


def _mm_t_body(h_ref, w_ref, o_ref):
    o_ref[...] = lax.dot_general(
        w_ref[...], h_ref[...], (((0,), (0,)), ((), ())),
        preferred_element_type=jnp.float32)


def _mm_t(ht, w):
    do = w.shape[1]
    return pl.pallas_call(
        _mm_t_body,
        out_shape=jax.ShapeDtypeStruct((do, N), jnp.float32),
    )(ht, w)


def _mm_t_pack_body(h_ref, we_ref, wo_ref, o_ref):
    dims = (((0,), (0,)), ((), ()))
    ye = lax.dot_general(we_ref[...], h_ref[...], dims,
                         preferred_element_type=jnp.float32)
    yo = lax.dot_general(wo_ref[...], h_ref[...], dims,
                         preferred_element_type=jnp.float32)
    o_ref[...] = _pack_pairs(ye, yo)


def _mm_t_pack(ht, we, wo):
    do2 = we.shape[1]
    return pl.pallas_call(
        _mm_t_pack_body,
        out_shape=jax.ShapeDtypeStruct((do2, N), jnp.int32),
    )(ht, we, wo)


def _comb_body(p_ref, o_ref):
    p = p_ref[...]
    s0 = jnp.sum(p[0:16], axis=0)
    s1 = jnp.sum(p[16:32], axis=0)
    s6 = jnp.concatenate([s0, s1], axis=0)
    i6 = (lax.broadcasted_iota(jnp.int32, (C, C), 0)
          == lax.broadcasted_iota(jnp.int32, (C, C), 1)).astype(jnp.float32)
    o_ref[...] = lax.dot_general(
        s6, i6, (((0,), (0,)), ((), ())), preferred_element_type=jnp.float32)


def _combine(parts):
    return pl.pallas_call(
        _comb_body,
        out_shape=jax.ShapeDtypeStruct((N, C), jnp.float32),
    )(parts)


def kernel(x, edge_index, W1, W2, W3):
    pe, h1p = _mm1_pack(edge_index, x, W1[:, 0::2], W1[:, 1::2])
    a1t = _agg_h(h1p, pe)
    h2p = _mm_t_pack(a1t, W2[:, 0::2], W2[:, 1::2])
    a2t = _agg_h(h2p, pe)
    h3t = _mm_t(a2t, W3)
    parts = _agg_out(h3t, pe)
    return _combine(parts)
